# baseline (speedup 1.0000x reference)
import jax
import jax.numpy as jnp
from jax import lax
from jax.experimental import pallas as pl
from jax.experimental.pallas import tpu as pltpu
from jax.experimental.pallas import tpu_sc as plsc

_TAIL = 3.0
_NB = 32
_MBW = 0.001
_MBH = 0.001
_BW = 1.0 - _MBW * _NB

_NC = 2
_NS = 16
_NW = _NC * _NS
_L = 16

_R = 512
_GROUPS = _R // _L

_LN2 = 0.6931471805599453
_SQRT2 = 1.4142135623730951


def _softlog(x):
    bits = lax.bitcast_convert_type(x, jnp.int32)
    e = lax.shift_right_logical(bits, 23) - 127
    m = lax.bitcast_convert_type(
        (bits & jnp.int32(0x007FFFFF)) | jnp.int32(0x3F800000), jnp.float32)
    big = m > _SQRT2
    m = jnp.where(big, m * 0.5, m)
    ef = e.astype(jnp.float32) + jnp.where(big, 1.0, 0.0)
    s = (m - 1.0) / (m + 1.0)
    z = s * s
    p = 2.0 + z * (0.6666666666 + z * (0.4 + z * 0.2857142857))
    return ef * _LN2 + s * p


def _qs_body(x_hbm, w_hbm, h_hbm, oo_hbm, ol_hbm,
             x_v0, w_v0, h_v0, x_v1, w_v1, h_v1, oo_v, ol_v,
             ew_s, eh_s, sem0, sem1):
    wid = lax.axis_index("s") * _NC + lax.axis_index("c")
    nd = x_hbm.shape[0]
    nr = x_hbm.shape[1]
    dpw = nd // _NW
    rchunks = nr // _R
    nch = dpw * rchunks
    iota = lax.iota(jnp.int32, _L)
    zeros = jnp.zeros((_L,), jnp.float32)
    bufs = ((x_v0, w_v0, h_v0, sem0), (x_v1, w_v1, h_v1, sem1))

    def _addr(ci):
        return wid * dpw + ci // rchunks, (ci % rchunks) * _R

    def _descs(ci, buf):
        x_v, w_v, h_v, sem = buf
        d, r0 = _addr(ci)
        return (
            pltpu.make_async_copy(w_hbm.at[d, :, pl.ds(r0, _R)], w_v, sem),
            pltpu.make_async_copy(h_hbm.at[:, d, pl.ds(r0, _R)], h_v, sem),
            pltpu.make_async_copy(x_hbm.at[d, pl.ds(r0, _R)], x_v, sem),
        )

    def _start(ci, buf):
        for desc in _descs(ci, buf):
            desc.start()

    def _wait(ci, buf):
        for desc in _descs(ci, buf):
            desc.wait()

    def _compute(ci, buf):
        x_v, w_v, h_v, _ = buf
        d, r0 = _addr(ci)

        @pl.loop(0, _GROUPS)
        def _group(g):
            rc = g * _L
            xin = x_v[pl.ds(rc, _L)]
            t = (jnp.clip(xin, -_TAIL, _TAIL) + _TAIL) * (1.0 / (2.0 * _TAIL))

            @plsc.parallel_loop(0, _NB, unroll=8, carry=zeros)
            def s1(j, acc):
                ew = jnp.exp(w_v[j, pl.ds(rc, _L)])
                ew_s[pl.ds(j * _L, _L)] = ew
                return acc + ew

            binv = _BW / s1

            eh0 = jnp.exp(h_v[0, pl.ds(rc, _L)])
            eh_s[pl.ds(0, _L)] = eh0

            @plsc.parallel_loop(1, _NB + 1, unroll=16,
                                carry=(zeros, zeros, iota * 0, eh0,
                                       zeros, zeros))
            def carry_hb(j, c):
                cl, cq, idx, ehp, loc_a, cq_a = c
                eh = jnp.exp(h_v[j, pl.ds(rc, _L)])
                eh_s[pl.ds(j * _L, _L)] = eh
                ew = ew_s[pl.ds(j * _L - _L, _L)]
                w = _MBW + binv * ew
                cl = cl + w
                d_ = (ehp + eh) * w
                cq = cq + d_
                take = (t >= cl) & (j < _NB)
                idx = idx + jnp.where(take, 1, 0)
                loc_a = loc_a + jnp.where(take, w, 0.0)
                cq_a = cq_a + jnp.where(take, d_, 0.0)
                return cl, cq, idx, eh, loc_a, cq_a

            _, cq_tot, idx, _eh_last, loc, cqa = carry_hb
            k2 = (1.0 - _MBH) / cq_tot
            hmb2 = 0.5 * _MBH

            lo16 = idx * _L + iota
            wat = _MBW + binv * plsc.load_gather(ew_s, [lo16])
            hl2 = hmb2 + k2 * plsc.load_gather(eh_s, [lo16])
            hr2 = hmb2 + k2 * plsc.load_gather(eh_s, [lo16 + _L])
            cdf = _MBH * loc + k2 * cqa

            alpha = (t - loc) / wat
            dh2 = hr2 - hl2
            out = wat * alpha * (dh2 * alpha + 2.0 * hl2) + cdf
            out = jnp.clip(out, 0.0, 1.0) * (2.0 * _TAIL) - _TAIL
            den = 2.0 * (alpha * dh2 + hl2)
            lad = _softlog(den)

            inside = (xin >= -_TAIL) & (xin <= _TAIL)
            oo_v[pl.ds(rc, _L)] = jnp.where(inside, out, xin)
            ol_v[pl.ds(rc, _L)] = jnp.where(inside, lad, 0.0)

        pltpu.sync_copy(oo_v, oo_hbm.at[d, pl.ds(r0, _R)])
        pltpu.sync_copy(ol_v, ol_hbm.at[d, pl.ds(r0, _R)])

    _start(0, bufs[0])

    @pl.loop(0, nch, step=2)
    def _chunk2(cj):
        _start(cj + 1, bufs[1])
        _wait(cj, bufs[0])
        _compute(cj, bufs[0])

        @pl.when(cj + 2 < nch)
        def _():
            _start(cj + 2, bufs[0])

        _wait(cj + 1, bufs[1])
        _compute(cj + 1, bufs[1])


def _make_qs(nd, nr):
    mesh = plsc.VectorSubcoreMesh(core_axis_name="c", subcore_axis_name="s",
                                  num_cores=_NC, num_subcores=_NS)
    return pl.kernel(
        _qs_body,
        out_type=(jax.ShapeDtypeStruct((nd, nr), jnp.float32),
                  jax.ShapeDtypeStruct((nd, nr), jnp.float32)),
        mesh=mesh,
        compiler_params=pltpu.CompilerParams(needs_layout_passes=False,
                                             use_tc_tiling_on_sc=True),
        scratch_types=[
            pltpu.VMEM((_R,), jnp.float32),
            pltpu.VMEM((_NB, _R), jnp.float32),
            pltpu.VMEM((_NB + 1, _R), jnp.float32),
            pltpu.VMEM((_R,), jnp.float32),
            pltpu.VMEM((_NB, _R), jnp.float32),
            pltpu.VMEM((_NB + 1, _R), jnp.float32),
            pltpu.VMEM((_R,), jnp.float32),
            pltpu.VMEM((_R,), jnp.float32),
            pltpu.VMEM((_NB * _L,), jnp.float32),
            pltpu.VMEM(((_NB + 1) * _L,), jnp.float32),
            pltpu.SemaphoreType.DMA,
            pltpu.SemaphoreType.DMA,
        ],
    )


@jax.jit
def kernel(x, w_, h_):
    n, d = x.shape
    qs = _make_qs(d, n)
    oo, ol = qs(x.T, w_.transpose(1, 2, 0), h_.transpose(2, 1, 0))
    return oo.T, ol.T


# speedup vs baseline: 1.1905x; 1.1905x over previous
import jax
import jax.numpy as jnp
from jax import lax
from jax.experimental import pallas as pl
from jax.experimental.pallas import tpu as pltpu
from jax.experimental.pallas import tpu_sc as plsc

_TAIL = 3.0
_NB = 32
_MBW = 0.001
_MBH = 0.001
_BW = 1.0 - _MBW * _NB

_NC = 2
_NS = 16
_NW = _NC * _NS
_L = 16

_R = 512
_GROUPS = _R // _L

_LN2 = 0.6931471805599453
_SQRT2 = 1.4142135623730951


def _softlog(x):
    bits = lax.bitcast_convert_type(x, jnp.int32)
    e = lax.shift_right_logical(bits, 23) - 127
    m = lax.bitcast_convert_type(
        (bits & jnp.int32(0x007FFFFF)) | jnp.int32(0x3F800000), jnp.float32)
    big = m > _SQRT2
    m = jnp.where(big, m * 0.5, m)
    ef = e.astype(jnp.float32) + jnp.where(big, 1.0, 0.0)
    s = (m - 1.0) / (m + 1.0)
    z = s * s
    p = 2.0 + z * (0.6666666666 + z * (0.4 + z * 0.2857142857))
    return ef * _LN2 + s * p


def _qs_body(x_hbm, w_hbm, h_hbm, oo_hbm, ol_hbm,
             x_v0, w_v0, h_v0, x_v1, w_v1, h_v1, oo_v, ol_v,
             ew_s, eh_s, l_s, q_s, sem0, sem1):
    wid = lax.axis_index("s") * _NC + lax.axis_index("c")
    nd = x_hbm.shape[0]
    nr = x_hbm.shape[1]
    dpw = nd // _NW
    rchunks = nr // _R
    nch = dpw * rchunks
    iota = lax.iota(jnp.int32, _L)
    zeros = jnp.zeros((_L,), jnp.float32)
    l_s[pl.ds(0, _L)] = zeros
    q_s[pl.ds(0, _L)] = zeros

    bufs = ((x_v0, w_v0, h_v0, sem0), (x_v1, w_v1, h_v1, sem1))

    def _addr(ci):
        return wid * dpw + ci // rchunks, (ci % rchunks) * _R

    def _descs(ci, buf):
        x_v, w_v, h_v, sem = buf
        d, r0 = _addr(ci)
        return (
            pltpu.make_async_copy(w_hbm.at[d, :, pl.ds(r0, _R)], w_v, sem),
            pltpu.make_async_copy(h_hbm.at[:, d, pl.ds(r0, _R)], h_v, sem),
            pltpu.make_async_copy(x_hbm.at[d, pl.ds(r0, _R)], x_v, sem),
        )

    def _start(ci, buf):
        for desc in _descs(ci, buf):
            desc.start()

    def _wait(ci, buf):
        for desc in _descs(ci, buf):
            desc.wait()

    def _compute(ci, buf):
        x_v, w_v, h_v, _ = buf
        d, r0 = _addr(ci)

        @pl.loop(0, _GROUPS)
        def _group(g):
            rc = g * _L
            xin = x_v[pl.ds(rc, _L)]
            t = (jnp.clip(xin, -_TAIL, _TAIL) + _TAIL) * (1.0 / (2.0 * _TAIL))

            @plsc.parallel_loop(0, _NB, unroll=8, carry=zeros)
            def s1(j, acc):
                ew = jnp.exp(w_v[j, pl.ds(rc, _L)])
                ew_s[pl.ds(j * _L, _L)] = ew
                return acc + ew

            binv = _BW / s1

            eh0 = jnp.exp(h_v[0, pl.ds(rc, _L)])
            eh_s[pl.ds(0, _L)] = eh0

            @plsc.parallel_loop(1, _NB + 1, unroll=16,
                                carry=(zeros, zeros, iota * 0, eh0))
            def carry_hb(j, c):
                cl, cq, idx, ehp = c
                eh = jnp.exp(h_v[j, pl.ds(rc, _L)])
                eh_s[pl.ds(j * _L, _L)] = eh
                ew = ew_s[pl.ds(j * _L - _L, _L)]
                w = _MBW + binv * ew
                cl = cl + w
                l_s[pl.ds(j * _L, _L)] = cl
                cq = cq + (ehp + eh) * w
                q_s[pl.ds(j * _L, _L)] = cq
                take = (t >= cl) & (j < _NB)
                idx = idx + jnp.where(take, 1, 0)
                return cl, cq, idx, eh

            _, cq_tot, idx, _eh_last = carry_hb
            k2 = (1.0 - _MBH) / cq_tot
            hmb2 = 0.5 * _MBH

            lo16 = idx * _L + iota
            loc = plsc.load_gather(l_s, [lo16])
            cqa = plsc.load_gather(q_s, [lo16])
            wat = _MBW + binv * plsc.load_gather(ew_s, [lo16])
            hl2 = hmb2 + k2 * plsc.load_gather(eh_s, [lo16])
            hr2 = hmb2 + k2 * plsc.load_gather(eh_s, [lo16 + _L])
            cdf = _MBH * loc + k2 * cqa

            alpha = (t - loc) / wat
            dh2 = hr2 - hl2
            out = wat * alpha * (dh2 * alpha + 2.0 * hl2) + cdf
            out = jnp.clip(out, 0.0, 1.0) * (2.0 * _TAIL) - _TAIL
            den = 2.0 * (alpha * dh2 + hl2)
            lad = _softlog(den)

            inside = (xin >= -_TAIL) & (xin <= _TAIL)
            oo_v[pl.ds(rc, _L)] = jnp.where(inside, out, xin)
            ol_v[pl.ds(rc, _L)] = jnp.where(inside, lad, 0.0)

        pltpu.sync_copy(oo_v, oo_hbm.at[d, pl.ds(r0, _R)])
        pltpu.sync_copy(ol_v, ol_hbm.at[d, pl.ds(r0, _R)])

    _start(0, bufs[0])

    @pl.loop(0, nch, step=2)
    def _chunk2(cj):
        _start(cj + 1, bufs[1])
        _wait(cj, bufs[0])
        _compute(cj, bufs[0])

        @pl.when(cj + 2 < nch)
        def _():
            _start(cj + 2, bufs[0])

        _wait(cj + 1, bufs[1])
        _compute(cj + 1, bufs[1])


def _make_qs(nd, nr):
    mesh = plsc.VectorSubcoreMesh(core_axis_name="c", subcore_axis_name="s",
                                  num_cores=_NC, num_subcores=_NS)
    return pl.kernel(
        _qs_body,
        out_type=(jax.ShapeDtypeStruct((nd, nr), jnp.float32),
                  jax.ShapeDtypeStruct((nd, nr), jnp.float32)),
        mesh=mesh,
        compiler_params=pltpu.CompilerParams(needs_layout_passes=False,
                                             use_tc_tiling_on_sc=True),
        scratch_types=[
            pltpu.VMEM((_R,), jnp.float32),
            pltpu.VMEM((_NB, _R), jnp.float32),
            pltpu.VMEM((_NB + 1, _R), jnp.float32),
            pltpu.VMEM((_R,), jnp.float32),
            pltpu.VMEM((_NB, _R), jnp.float32),
            pltpu.VMEM((_NB + 1, _R), jnp.float32),
            pltpu.VMEM((_R,), jnp.float32),
            pltpu.VMEM((_R,), jnp.float32),
            pltpu.VMEM((_NB * _L,), jnp.float32),
            pltpu.VMEM(((_NB + 1) * _L,), jnp.float32),
            pltpu.VMEM(((_NB + 1) * _L,), jnp.float32),
            pltpu.VMEM(((_NB + 1) * _L,), jnp.float32),
            pltpu.SemaphoreType.DMA,
            pltpu.SemaphoreType.DMA,
        ],
    )


@jax.jit
def kernel(x, w_, h_):
    n, d = x.shape
    qs = _make_qs(d, n)
    oo, ol = qs(x.T, w_.transpose(1, 2, 0), h_.transpose(2, 1, 0))
    return oo.T, ol.T
